# baseline (speedup 1.0000x reference)
import jax
import jax.numpy as jnp
from jax.experimental import pallas as pl
from jax.experimental.pallas import tpu as pltpu


_BLOCK_ROWS = 1024


def _copy_kernel(x_ref, o_ref):
    o_ref[...] = x_ref[...]


def kernel(x):
    rows, cols = x.shape
    return pl.pallas_call(
        _copy_kernel,
        grid=(rows // _BLOCK_ROWS,),
        in_specs=[pl.BlockSpec((_BLOCK_ROWS, cols), lambda i: (i, 0))],
        out_specs=pl.BlockSpec((_BLOCK_ROWS, cols), lambda i: (i, 0)),
        out_shape=jax.ShapeDtypeStruct(x.shape, x.dtype),
        compiler_params=pltpu.CompilerParams(
            dimension_semantics=("parallel",),
        ),
    )(x)


# speedup vs baseline: 1.5582x; 1.5582x over previous
import jax
import jax.numpy as jnp
from jax.experimental import pallas as pl
from jax.experimental.pallas import tpu as pltpu


_N_CHUNKS = 16


def _copy_kernel(x_ref, o_ref, buf, sem_in, sem_out):
    chunk = x_ref.shape[0] // _N_CHUNKS
    ins = [
        pltpu.make_async_copy(
            x_ref.at[pl.ds(c * chunk, chunk)],
            buf.at[pl.ds(c * chunk, chunk)],
            sem_in.at[c],
        )
        for c in range(_N_CHUNKS)
    ]
    outs = [
        pltpu.make_async_copy(
            buf.at[pl.ds(c * chunk, chunk)],
            o_ref.at[pl.ds(c * chunk, chunk)],
            sem_out.at[c],
        )
        for c in range(_N_CHUNKS)
    ]
    for cp in ins:
        cp.start()
    for c in range(_N_CHUNKS):
        ins[c].wait()
        outs[c].start()
    for cp in outs:
        cp.wait()


def kernel(x):
    return pl.pallas_call(
        _copy_kernel,
        out_shape=jax.ShapeDtypeStruct(x.shape, x.dtype),
        in_specs=[pl.BlockSpec(memory_space=pl.ANY)],
        out_specs=pl.BlockSpec(memory_space=pl.ANY),
        scratch_shapes=[
            pltpu.VMEM(x.shape, x.dtype),
            pltpu.SemaphoreType.DMA((_N_CHUNKS,)),
            pltpu.SemaphoreType.DMA((_N_CHUNKS,)),
        ],
    )(x)
